# baseline (speedup 1.0000x reference)
import functools

import jax
import jax.numpy as jnp
from jax import lax
from jax.experimental import pallas as pl
from jax.experimental.pallas import tpu as pltpu
from jax.experimental.pallas import tpu_sc as plsc

_INFO = plsc.get_sparse_core_info()
_NC, _NS = _INFO.num_cores, _INFO.num_subcores
_NW = _NC * _NS

_B, _T = 16384, 200
_D = 64
_C = 128
_ROWS = (_B * _T) // _C
_ROWS_PER_W = _ROWS // _NW
_K = 8
_N_SLABS = _ROWS_PER_W // _K


@functools.partial(
    pl.kernel,
    out_type=jax.ShapeDtypeStruct((_ROWS, _C, _D), jnp.float32),
    mesh=plsc.VectorSubcoreMesh(core_axis_name="c", subcore_axis_name="s"),
    scratch_types=[
        pltpu.VMEM((_K, _C), jnp.int32),
        pltpu.VMEM((_K, _C, _D), jnp.float32),
        pltpu.SemaphoreType.DMA,
    ],
    compiler_params=pltpu.CompilerParams(use_tc_tiling_on_sc=False),
)
def _gather_kernel(idx_hbm, table_hbm, out_hbm, idx_v, rows_v, sem):
    wid = lax.axis_index("s") * _NC + lax.axis_index("c")
    row0 = wid * _ROWS_PER_W

    def slab(s, _):
        base = row0 + s * _K
        pltpu.sync_copy(idx_hbm.at[pl.ds(base, _K)], idx_v)
        handles = []
        for j in range(_K):
            handles.append(
                pltpu.async_copy(table_hbm.at[idx_v.at[j]], rows_v.at[j], sem)
            )
        for h in handles:
            h.wait()
        pltpu.sync_copy(rows_v, out_hbm.at[pl.ds(base, _K)])
        return 0

    lax.fori_loop(0, _N_SLABS, slab, 0)


def kernel(data, tvectors):
    idx = data.astype(jnp.int32).reshape(_ROWS, _C)
    out = _gather_kernel(idx, tvectors)
    return out.reshape(_B, _T, _D)


# speedup vs baseline: 1.0102x; 1.0102x over previous
import functools

import jax
import jax.numpy as jnp
from jax import lax
from jax.experimental import pallas as pl
from jax.experimental.pallas import tpu as pltpu
from jax.experimental.pallas import tpu_sc as plsc

_INFO = plsc.get_sparse_core_info()
_NC, _NS = _INFO.num_cores, _INFO.num_subcores
_NW = _NC * _NS

_B, _T = 16384, 200
_D = 64
_C = 128
_ROWS = (_B * _T) // _C
_ROWS_PER_W = _ROWS // _NW
_K = 4
_N_SLABS = _ROWS_PER_W // _K


@functools.partial(
    pl.kernel,
    out_type=jax.ShapeDtypeStruct((_ROWS, _C, _D), jnp.float32),
    mesh=plsc.VectorSubcoreMesh(core_axis_name="c", subcore_axis_name="s"),
    scratch_types=[
        pltpu.VMEM((2, _K, _C), jnp.int32),
        pltpu.VMEM((2, _K, _C, _D), jnp.float32),
        pltpu.SemaphoreType.DMA,
        pltpu.SemaphoreType.DMA,
        pltpu.SemaphoreType.DMA,
    ],
    compiler_params=pltpu.CompilerParams(use_tc_tiling_on_sc=False),
)
def _gather_kernel(idx_hbm, table_hbm, out_hbm, idx_v, rows_v, sem_g,
                   sem_o0, sem_o1):
    wid = lax.axis_index("s") * _NC + lax.axis_index("c")
    row0 = wid * _ROWS_PER_W
    sem_o = (sem_o0, sem_o1)

    def run_slab(t, b, drain_out):
        base = row0 + t * _K
        pltpu.sync_copy(idx_hbm.at[pl.ds(base, _K)], idx_v.at[b])
        if drain_out:
            pltpu.make_async_copy(
                rows_v.at[b], out_hbm.at[pl.ds(base, _K)], sem_o[b]
            ).wait()
        handles = [
            pltpu.async_copy(
                table_hbm.at[idx_v.at[b].at[j]], rows_v.at[b].at[j], sem_g
            )
            for j in range(_K)
        ]
        for h in handles:
            h.wait()
        pltpu.async_copy(rows_v.at[b], out_hbm.at[pl.ds(base, _K)], sem_o[b])

    run_slab(0, 0, drain_out=False)
    run_slab(1, 1, drain_out=False)

    def body(i, _):
        s = 2 + 2 * i
        run_slab(s, 0, drain_out=True)
        run_slab(s + 1, 1, drain_out=True)
        return 0

    lax.fori_loop(0, (_N_SLABS - 2) // 2, body, 0, unroll=False)

    for b in range(2):
        pltpu.make_async_copy(
            rows_v.at[b], out_hbm.at[pl.ds(row0, _K)], sem_o[b]
        ).wait()


def kernel(data, tvectors):
    idx = data.astype(jnp.int32).reshape(_ROWS, _C)
    out = _gather_kernel(idx, tvectors)
    return out.reshape(_B, _T, _D)


# speedup vs baseline: 1.0342x; 1.0238x over previous
import functools

import jax
import jax.numpy as jnp
from jax import lax
from jax.experimental import pallas as pl
from jax.experimental.pallas import tpu as pltpu
from jax.experimental.pallas import tpu_sc as plsc

_INFO = plsc.get_sparse_core_info()
_NC, _NS = _INFO.num_cores, _INFO.num_subcores
_NW = _NC * _NS

_B, _T = 16384, 200
_D = 64
_C = 128
_ROWS = (_B * _T) // _C
_ROWS_PER_W = _ROWS // _NW
_K = 4
_N_SLABS = _ROWS_PER_W // _K


@functools.partial(
    pl.kernel,
    out_type=jax.ShapeDtypeStruct((_ROWS, _C, _D), jnp.float32),
    mesh=plsc.VectorSubcoreMesh(core_axis_name="c", subcore_axis_name="s"),
    scratch_types=[
        pltpu.VMEM((2, _K, _C), jnp.int32),
        pltpu.VMEM((2, _K, _C, _D), jnp.float32),
        pltpu.SemaphoreType.DMA,
        pltpu.SemaphoreType.DMA,
        pltpu.SemaphoreType.DMA,
        pltpu.SemaphoreType.DMA,
        pltpu.SemaphoreType.DMA,
        pltpu.SemaphoreType.DMA,
    ],
    compiler_params=pltpu.CompilerParams(use_tc_tiling_on_sc=False),
)
def _gather_kernel(idx_hbm, table_hbm, out_hbm, idx_v, rows_v,
                   sem_i0, sem_i1, sem_g0, sem_g1, sem_o0, sem_o1):
    wid = lax.axis_index("s") * _NC + lax.axis_index("c")
    row0 = wid * _ROWS_PER_W
    sem_i, sem_g, sem_o = (sem_i0, sem_i1), (sem_g0, sem_g1), (sem_o0, sem_o1)

    def base(t):
        return row0 + t * _K

    def fire_gathers(t, b):
        for j in range(_K):
            pltpu.async_copy(
                table_hbm.at[idx_v.at[b].at[j]], rows_v.at[b].at[j], sem_g[b]
            )

    def drain_gathers(t, b):
        pltpu.make_async_copy(
            out_hbm.at[pl.ds(base(t), _K)], rows_v.at[b], sem_g[b]
        ).wait()

    def writeback(t, b):
        pltpu.async_copy(rows_v.at[b], out_hbm.at[pl.ds(base(t), _K)], sem_o[b])

    def drain_writeback(t, b):
        pltpu.make_async_copy(
            rows_v.at[b], out_hbm.at[pl.ds(base(t), _K)], sem_o[b]
        ).wait()

    def prefetch_idx(t, b):
        pltpu.async_copy(idx_hbm.at[pl.ds(base(t), _K)], idx_v.at[b], sem_i[b])

    def wait_idx(t, b):
        pltpu.make_async_copy(
            idx_hbm.at[pl.ds(base(t), _K)], idx_v.at[b], sem_i[b]
        ).wait()

    pltpu.sync_copy(idx_hbm.at[pl.ds(base(0), _K)], idx_v.at[0])
    fire_gathers(0, 0)
    prefetch_idx(1, 1)
    wait_idx(1, 1)
    fire_gathers(1, 1)
    drain_gathers(0, 0)
    writeback(0, 0)
    prefetch_idx(2, 0)

    def body(i, _):
        s = 2 + 2 * i
        for b in range(2):
            t = s + b
            wait_idx(t, b)
            drain_writeback(t - 2, b)
            fire_gathers(t, b)
            drain_gathers(t - 1, 1 - b)
            writeback(t - 1, 1 - b)
            tn = jnp.minimum(t + 1, _N_SLABS - 1)
            prefetch_idx(tn, 1 - b)
        return 0

    lax.fori_loop(0, (_N_SLABS - 2) // 2, body, 0, unroll=False)

    last = _N_SLABS - 1
    wait_idx(last, 0)
    drain_gathers(last, 1)
    writeback(last, 1)
    drain_writeback(last - 1, 0)
    drain_writeback(last, 1)


def kernel(data, tvectors):
    idx = data.astype(jnp.int32).reshape(_ROWS, _C)
    out = _gather_kernel(idx, tvectors)
    return out.reshape(_B, _T, _D)
